# Initial kernel scaffold; baseline (speedup 1.0000x reference)
import jax
import jax.numpy as jnp
from jax.experimental import pallas as pl


def kernel(x, lens, output_mask):
    raise NotImplementedError("write your pallas kernel here")


# speedup vs baseline: 1.3664x; 1.3664x over previous
import jax
import jax.numpy as jnp
from jax.experimental import pallas as pl
from jax.experimental.pallas import tpu as pltpu

_BS = 512


def _body(lens_ref, x_ref, mask_ref, o_ref):
    b = pl.program_id(0)
    s = pl.program_id(1)
    first_row = s * _BS
    rows = first_row + jax.lax.broadcasted_iota(jnp.int32, (_BS, 1), 0)
    pad = rows >= lens_ref[b]
    o_ref[0] = jnp.where(pad, mask_ref[0][None, :], x_ref[0])


def _x_map(b, s, lens_ref):
    last_live = jnp.maximum(jax.lax.div(lens_ref[b] + (_BS - 1), _BS) - 1, 0)
    return (b, jnp.minimum(s, last_live), 0)


def kernel(x, lens, output_mask):
    B, S, F = x.shape
    lens_i = lens.astype(jnp.int32)
    mask2 = output_mask.reshape(1, F)
    grid_spec = pltpu.PrefetchScalarGridSpec(
        num_scalar_prefetch=1,
        grid=(B, S // _BS),
        in_specs=[
            pl.BlockSpec((1, _BS, F), _x_map),
            pl.BlockSpec((1, F), lambda b, s, lens_ref: (0, 0)),
        ],
        out_specs=pl.BlockSpec((1, _BS, F), lambda b, s, lens_ref: (b, s, 0)),
    )
    return pl.pallas_call(
        _body,
        grid_spec=grid_spec,
        out_shape=jax.ShapeDtypeStruct((B, S, F), x.dtype),
    )(lens_i, x, mask2)


# speedup vs baseline: 1.5244x; 1.1156x over previous
import jax
import jax.numpy as jnp
from jax.experimental import pallas as pl
from jax.experimental.pallas import tpu as pltpu

_BS = 1024


def _body(lens_ref, x_ref, mask_ref, o_ref):
    b = pl.program_id(0)
    s = pl.program_id(1)
    first_row = s * _BS
    rows = first_row + jax.lax.broadcasted_iota(jnp.int32, (_BS, 1), 0)
    pad = rows >= lens_ref[b]
    o_ref[0] = jnp.where(pad, mask_ref[0][None, :], x_ref[0])


def _x_map(b, s, lens_ref):
    last_live = jnp.maximum(jax.lax.div(lens_ref[b] + (_BS - 1), _BS) - 1, 0)
    return (b, jnp.minimum(s, last_live), 0)


def kernel(x, lens, output_mask):
    B, S, F = x.shape
    lens_i = lens.astype(jnp.int32)
    mask2 = output_mask.reshape(1, F)
    grid_spec = pltpu.PrefetchScalarGridSpec(
        num_scalar_prefetch=1,
        grid=(B, S // _BS),
        in_specs=[
            pl.BlockSpec((1, _BS, F), _x_map),
            pl.BlockSpec((1, F), lambda b, s, lens_ref: (0, 0)),
        ],
        out_specs=pl.BlockSpec((1, _BS, F), lambda b, s, lens_ref: (b, s, 0)),
    )
    return pl.pallas_call(
        _body,
        grid_spec=grid_spec,
        out_shape=jax.ShapeDtypeStruct((B, S, F), x.dtype),
    )(lens_i, x, mask2)
